# Initial kernel scaffold; baseline (speedup 1.0000x reference)
import jax
import jax.numpy as jnp
from jax.experimental import pallas as pl


def kernel(relative):
    raise NotImplementedError("write your pallas kernel here")


# speedup vs baseline: 10.5681x; 10.5681x over previous
import jax
import jax.numpy as jnp
from jax import lax
from jax.experimental import pallas as pl
from jax.experimental.pallas import tpu as pltpu
from jax.experimental.pallas import tpu_sc as plsc

DIM = 512
DIM_KQ = 8
DIM_V = 16
CHAN = 2 * DIM_KQ + DIM_V
TBL = 2 * DIM - 1
LANES = 16
NC, NS = 2, 16
BLK_ROWS = 64
N_BLKS = DIM // BLK_ROWS
CHUNKS = DIM // LANES


def _body(rel_hbm, q_hbm, k_hbm, v_hbm, tbl_v, buf_v, sem0, sem1):
    wid = lax.axis_index("s") * NC + lax.axis_index("c")
    pltpu.sync_copy(rel_hbm.at[wid], tbl_v)
    iota = lax.iota(jnp.int32, LANES)

    def run(out_hbm, base):
        cc = wid - base
        sems = (sem0, sem1)

        def build(b, slot):
            def row_body(r, carry):
                base_idx = (DIM - 1) + b * BLK_ROWS + r
                for kk in range(CHUNKS):
                    idx = (base_idx - kk * LANES) - iota
                    vals = plsc.load_gather(tbl_v, [idx])
                    buf_v[slot, r, pl.ds(kk * LANES, LANES)] = vals
                return carry
            lax.fori_loop(0, BLK_ROWS, row_body, 0)

        copies = [None, None]
        for b in range(N_BLKS):
            s = b & 1
            if copies[s] is not None:
                copies[s].wait()
            build(b, s)
            copies[s] = pltpu.async_copy(
                buf_v.at[s],
                out_hbm.at[cc, pl.ds(b * BLK_ROWS, BLK_ROWS)],
                sems[s],
            )
        copies[0].wait()
        copies[1].wait()

    @pl.when(wid < DIM_KQ)
    def _():
        run(q_hbm, 0)

    @pl.when((wid >= DIM_KQ) & (wid < 2 * DIM_KQ))
    def _():
        run(k_hbm, DIM_KQ)

    @pl.when(wid >= 2 * DIM_KQ)
    def _():
        run(v_hbm, 2 * DIM_KQ)


def kernel(relative):
    f = pl.kernel(
        _body,
        out_type=(
            jax.ShapeDtypeStruct((DIM_KQ, DIM, DIM), jnp.float32),
            jax.ShapeDtypeStruct((DIM_KQ, DIM, DIM), jnp.float32),
            jax.ShapeDtypeStruct((DIM_V, DIM, DIM), jnp.float32),
        ),
        mesh=plsc.VectorSubcoreMesh(
            core_axis_name="c", subcore_axis_name="s",
            num_cores=NC, num_subcores=NS,
        ),
        scratch_types=[
            pltpu.VMEM((TBL,), jnp.float32),
            pltpu.VMEM((2, BLK_ROWS, DIM), jnp.float32),
            pltpu.SemaphoreType.DMA,
            pltpu.SemaphoreType.DMA,
        ],
        compiler_params=pltpu.CompilerParams(needs_layout_passes=False),
    )
    return f(relative)


# speedup vs baseline: 19.4471x; 1.8402x over previous
import jax
import jax.numpy as jnp
from jax import lax
from jax.experimental import pallas as pl
from jax.experimental.pallas import tpu as pltpu
from jax.experimental.pallas import tpu_sc as plsc

DIM = 512
DIM_KQ = 8
DIM_V = 16
CHAN = 2 * DIM_KQ + DIM_V
TBL = 2 * DIM - 1
LANES = 16
NC, NS = 2, 16
BLK_ROWS = 64
N_BLKS = DIM // BLK_ROWS
CHUNKS = DIM // LANES
ROW_UNROLL = 4


def _body(rel_hbm, q_hbm, k_hbm, v_hbm, tbl_v, buf_v, sem0, sem1):
    wid = lax.axis_index("s") * NC + lax.axis_index("c")
    pltpu.sync_copy(rel_hbm.at[wid], tbl_v)
    riota = (DIM - 1) - lax.iota(jnp.int32, LANES)
    sems = (sem0, sem1)

    def build(b, slot):
        @plsc.parallel_loop(0, BLK_ROWS, 1, unroll=ROW_UNROLL)
        def _row(r):
            base = b * BLK_ROWS + r
            for kk in range(CHUNKS):
                idx = riota + (base - kk * LANES)
                vals = plsc.load_gather(tbl_v, [idx])
                buf_v[slot, r, pl.ds(kk * LANES, LANES)] = vals

    def start(b, slot):
        rows = pl.ds(b * BLK_ROWS, BLK_ROWS)

        @pl.when(wid < DIM_KQ)
        def _():
            pltpu.async_copy(buf_v.at[slot], q_hbm.at[wid, rows], sems[slot])

        @pl.when((wid >= DIM_KQ) & (wid < 2 * DIM_KQ))
        def _():
            pltpu.async_copy(buf_v.at[slot], k_hbm.at[wid - DIM_KQ, rows],
                             sems[slot])

        @pl.when(wid >= 2 * DIM_KQ)
        def _():
            pltpu.async_copy(buf_v.at[slot], v_hbm.at[wid - 2 * DIM_KQ, rows],
                             sems[slot])

    def drain(slot):
        pltpu.make_async_copy(
            buf_v.at[slot], q_hbm.at[0, pl.ds(0, BLK_ROWS)], sems[slot]
        ).wait()

    for b in range(N_BLKS):
        s = b & 1
        if b >= 2:
            drain(s)
        build(b, s)
        start(b, s)
    drain(0)
    drain(1)


def kernel(relative):
    f = pl.kernel(
        _body,
        out_type=(
            jax.ShapeDtypeStruct((DIM_KQ, DIM, DIM), jnp.float32),
            jax.ShapeDtypeStruct((DIM_KQ, DIM, DIM), jnp.float32),
            jax.ShapeDtypeStruct((DIM_V, DIM, DIM), jnp.float32),
        ),
        mesh=plsc.VectorSubcoreMesh(
            core_axis_name="c", subcore_axis_name="s",
            num_cores=NC, num_subcores=NS,
        ),
        scratch_types=[
            pltpu.VMEM((TBL,), jnp.float32),
            pltpu.VMEM((2, BLK_ROWS, DIM), jnp.float32),
            pltpu.SemaphoreType.DMA,
            pltpu.SemaphoreType.DMA,
        ],
        compiler_params=pltpu.CompilerParams(needs_layout_passes=False),
    )
    return f(relative)


# speedup vs baseline: 24.1047x; 1.2395x over previous
import jax
import jax.numpy as jnp
from jax import lax
from jax.experimental import pallas as pl
from jax.experimental.pallas import tpu as pltpu
from jax.experimental.pallas import tpu_sc as plsc

DIM = 512
DIM_KQ = 8
DIM_V = 16
CHAN = 2 * DIM_KQ + DIM_V
TBL = 2 * DIM - 1
LANES = 16
NC, NS = 2, 16
BLK_ROWS = 64
N_BLKS = DIM // BLK_ROWS
CHUNKS = DIM // LANES
ROW_UNROLL = 8


def _body(rel_hbm, q_hbm, k_hbm, v_hbm, tbl_v, buf_v, sem0, sem1):
    wid = lax.axis_index("s") * NC + lax.axis_index("c")
    pltpu.sync_copy(rel_hbm.at[wid], tbl_v)
    riota = (DIM - 1) - lax.iota(jnp.int32, LANES)
    sems = (sem0, sem1)

    def build(b, slot):
        @plsc.parallel_loop(0, BLK_ROWS, 1, unroll=ROW_UNROLL)
        def _row(r):
            base = b * BLK_ROWS + r
            for kk in range(CHUNKS):
                idx = riota + (base - kk * LANES)
                vals = plsc.load_gather(tbl_v, [idx])
                buf_v[slot, r, pl.ds(kk * LANES, LANES)] = vals

    def start(b, slot):
        rows = pl.ds(b * BLK_ROWS, BLK_ROWS)

        @pl.when(wid < DIM_KQ)
        def _():
            pltpu.async_copy(buf_v.at[slot], q_hbm.at[wid, rows], sems[slot])

        @pl.when((wid >= DIM_KQ) & (wid < 2 * DIM_KQ))
        def _():
            pltpu.async_copy(buf_v.at[slot], k_hbm.at[wid - DIM_KQ, rows],
                             sems[slot])

        @pl.when(wid >= 2 * DIM_KQ)
        def _():
            pltpu.async_copy(buf_v.at[slot], v_hbm.at[wid - 2 * DIM_KQ, rows],
                             sems[slot])

    def drain(slot):
        pltpu.make_async_copy(
            buf_v.at[slot], q_hbm.at[0, pl.ds(0, BLK_ROWS)], sems[slot]
        ).wait()

    for b in range(N_BLKS):
        s = b & 1
        if b >= 2:
            drain(s)
        build(b, s)
        start(b, s)
    drain(0)
    drain(1)


def kernel(relative):
    f = pl.kernel(
        _body,
        out_type=(
            jax.ShapeDtypeStruct((DIM_KQ, DIM, DIM), jnp.float32),
            jax.ShapeDtypeStruct((DIM_KQ, DIM, DIM), jnp.float32),
            jax.ShapeDtypeStruct((DIM_V, DIM, DIM), jnp.float32),
        ),
        mesh=plsc.VectorSubcoreMesh(
            core_axis_name="c", subcore_axis_name="s",
            num_cores=NC, num_subcores=NS,
        ),
        scratch_types=[
            pltpu.VMEM((TBL,), jnp.float32),
            pltpu.VMEM((2, BLK_ROWS, DIM), jnp.float32),
            pltpu.SemaphoreType.DMA,
            pltpu.SemaphoreType.DMA,
        ],
        compiler_params=pltpu.CompilerParams(needs_layout_passes=False),
    )
    return f(relative)
